# baseline (speedup 1.0000x reference)
import functools

import numpy as np
import jax
import jax.numpy as jnp
from jax import lax
from jax.experimental import pallas as pl
from jax.experimental.pallas import tpu as pltpu
from jax.experimental.pallas import tpu_sc as plsc

B, C, H, W = 32, 384, 24, 24
S = H * W
N = B * C
_RECIP = np.float32(1.0 / 576.0)


def _stats_kernel(y3_ref, gidx_ref):
    xb = y3_ref[0]

    acc = None
    for k in range(3):
        for t in range(24):
            row = 24 * t + 8 * k
            sl = xb[row : row + 8, :]
            acc = sl if acc is None else acc + sl
    u0 = acc[0:1, :] + acc[4:5, :]
    u2 = acc[2:3, :] + acc[6:7, :]
    u1 = acc[1:2, :] + acc[5:6, :]
    u3 = acc[3:4, :] + acc[7:8, :]
    total = (u0 + u2) + (u1 + u3)
    m = total * _RECIP

    ik = lax.bitcast_convert_type(m, jnp.int32)
    key = jnp.where(ik >= 0, ik, ik ^ jnp.int32(0x7FFFFFFF))
    key_t = jnp.transpose(key)

    ii = lax.broadcasted_iota(jnp.int32, (C, C), 0)
    jj = lax.broadcasted_iota(jnp.int32, (C, C), 1)
    kj = jnp.broadcast_to(key_t, (C, C))
    ki = jnp.broadcast_to(key, (C, C))
    contrib = (kj > ki) | ((kj == ki) & (ii < jj))
    rank = jnp.sum(contrib.astype(jnp.int32), axis=0, keepdims=True)

    rank_t = jnp.transpose(rank)
    onehot = jnp.broadcast_to(rank_t, (C, C)) == jj
    idx = jnp.sum(jnp.where(onehot, ii, 0), axis=0, keepdims=True)
    gidx_ref[0] = idx


_stats = pl.pallas_call(
    _stats_kernel,
    grid=(B,),
    in_specs=[pl.BlockSpec((1, S, C), lambda b: (b, 0, 0))],
    out_specs=pl.BlockSpec((1, 1, C), lambda b: (b, 0, 0)),
    out_shape=jax.ShapeDtypeStruct((B, 1, C), jnp.int32),
)

_info = plsc.get_sparse_core_info()
_NC = _info.num_cores
_NW = _info.num_cores * _info.num_subcores
_L = _info.num_lanes
_CHS = 64
_NCHUNK = S // _CHS
_NG = C // _L


@functools.partial(
    pl.kernel,
    mesh=plsc.VectorSubcoreMesh(core_axis_name="c", subcore_axis_name="s"),
    compiler_params=pltpu.CompilerParams(needs_layout_passes=False),
    out_type=jax.ShapeDtypeStruct((B, S, C), jnp.float32),
    scratch_types=[
        pltpu.VMEM((1, C), jnp.int32),
        pltpu.VMEM((_CHS, C), jnp.float32),
        pltpu.VMEM((_CHS, C), jnp.float32),
        pltpu.VMEM((_CHS, C), jnp.float32),
        pltpu.VMEM((_CHS, C), jnp.float32),
        pltpu.SemaphoreType.DMA,
        pltpu.SemaphoreType.DMA,
        pltpu.SemaphoreType.DMA,
        pltpu.SemaphoreType.DMA,
    ],
)
def _permute(x_hbm, idx_hbm, out_hbm, idx_v, inb0, inb1, outb0, outb1,
             sin0, sin1, sout0, sout1):
    b = lax.axis_index("s") * _NC + lax.axis_index("c")
    pltpu.sync_copy(idx_hbm.at[pl.ds(b, 1)], idx_v)
    idx16 = [idx_v[0, pl.ds(_L * g, _L)] for g in range(_NG)]
    inbs, outbs = [inb0, inb1], [outb0, outb1]
    sins, souts = [sin0, sin1], [sout0, sout1]
    h_in = [None, None]
    h_out = [None, None]
    h_in[0] = pltpu.async_copy(x_hbm.at[b, pl.ds(0, _CHS)], inbs[0], sins[0])
    for ch in range(_NCHUNK):
        p = ch % 2
        q = 1 - p
        if ch + 1 < _NCHUNK:
            h_in[q] = pltpu.async_copy(
                x_hbm.at[b, pl.ds((ch + 1) * _CHS, _CHS)], inbs[q], sins[q]
            )
        h_in[p].wait()
        if ch >= 2:
            h_out[p].wait()
        inb, outb = inbs[p], outbs[p]

        @plsc.parallel_loop(0, _CHS, unroll=8)
        def body(s):
            srow = jnp.full((_L,), s, jnp.int32)
            for g in range(_NG):
                vals = plsc.load_gather(inb, [srow, idx16[g]])
                outb[s, pl.ds(_L * g, _L)] = vals
        h_out[p] = pltpu.async_copy(
            outb, out_hbm.at[b, pl.ds(ch * _CHS, _CHS)], souts[p]
        )
    h_out[(_NCHUNK - 2) % 2].wait()
    h_out[(_NCHUNK - 1) % 2].wait()


def kernel(x):
    x3 = jnp.transpose(x.reshape(B, C, S), (0, 2, 1))
    lidx = _stats(x3).reshape(B, C)
    out3 = _permute(x3, lidx)
    return jnp.transpose(out3, (0, 2, 1)).reshape(B, C, H, W)


# speedup vs baseline: 1.1843x; 1.1843x over previous
import functools

import numpy as np
import jax
import jax.numpy as jnp
from jax import lax
from jax.experimental import pallas as pl
from jax.experimental.pallas import tpu as pltpu
from jax.experimental.pallas import tpu_sc as plsc

B, C, H, W = 32, 384, 24, 24
S = H * W
N = B * C
_RECIP = np.float32(1.0 / 576.0)


def _stats_kernel(y3_ref, gidx_ref):
    xb = y3_ref[0]

    acc = None
    for k in range(3):
        for t in range(24):
            row = 24 * t + 8 * k
            sl = xb[row : row + 8, :]
            acc = sl if acc is None else acc + sl
    u0 = acc[0:1, :] + acc[4:5, :]
    u2 = acc[2:3, :] + acc[6:7, :]
    u1 = acc[1:2, :] + acc[5:6, :]
    u3 = acc[3:4, :] + acc[7:8, :]
    total = (u0 + u2) + (u1 + u3)
    m = total * _RECIP

    ik = lax.bitcast_convert_type(m, jnp.int32)
    key = jnp.where(ik >= 0, ik, ik ^ jnp.int32(0x7FFFFFFF))
    key_t = jnp.transpose(key)

    ii = lax.broadcasted_iota(jnp.int32, (C, C), 0)
    jj = lax.broadcasted_iota(jnp.int32, (C, C), 1)
    kj = jnp.broadcast_to(key_t, (C, C))
    ki = jnp.broadcast_to(key, (C, C))
    contrib = (kj > ki) | ((kj == ki) & (ii < jj))
    rank = jnp.sum(contrib.astype(jnp.int32), axis=0, keepdims=True)

    rank_t = jnp.transpose(rank)
    onehot = jnp.broadcast_to(rank_t, (C, C)) == jj
    idx = jnp.sum(jnp.where(onehot, ii, 0), axis=0, keepdims=True)
    gidx_ref[0] = idx


_stats = pl.pallas_call(
    _stats_kernel,
    grid=(B,),
    in_specs=[pl.BlockSpec((1, S, C), lambda b: (b, 0, 0))],
    out_specs=pl.BlockSpec((1, 1, C), lambda b: (b, 0, 0)),
    out_shape=jax.ShapeDtypeStruct((B, 1, C), jnp.int32),
)

_info = plsc.get_sparse_core_info()
_NC = _info.num_cores
_NW = _info.num_cores * _info.num_subcores
_L = _info.num_lanes
_CHS = 64
_NCHUNK = S // _CHS
_NG = C // _L


@functools.partial(
    pl.kernel,
    mesh=plsc.VectorSubcoreMesh(core_axis_name="c", subcore_axis_name="s"),
    compiler_params=pltpu.CompilerParams(needs_layout_passes=False),
    out_type=jax.ShapeDtypeStruct((B, S, C), jnp.float32),
    scratch_types=[
        pltpu.VMEM((1, C), jnp.int32),
        pltpu.VMEM((_CHS, C), jnp.float32),
        pltpu.VMEM((_CHS, C), jnp.float32),
        pltpu.VMEM((_CHS, C), jnp.float32),
        pltpu.VMEM((_CHS, C), jnp.float32),
        pltpu.SemaphoreType.DMA,
        pltpu.SemaphoreType.DMA,
        pltpu.SemaphoreType.DMA,
        pltpu.SemaphoreType.DMA,
    ],
)
def _permute(x_hbm, idx_hbm, out_hbm, idx_v, inb0, inb1, outb0, outb1,
             sin0, sin1, sout0, sout1):
    b = lax.axis_index("s") * _NC + lax.axis_index("c")
    pltpu.sync_copy(idx_hbm.at[pl.ds(b, 1)], idx_v)
    idx16 = [idx_v[0, pl.ds(_L * g, _L)] for g in range(_NG)]
    inbs, outbs = [inb0, inb1], [outb0, outb1]
    sins, souts = [sin0, sin1], [sout0, sout1]
    h_in = [None, None]
    h_out = [None, None]
    h_in[0] = pltpu.async_copy(x_hbm.at[b, pl.ds(0, _CHS)], inbs[0], sins[0])
    for ch in range(_NCHUNK):
        p = ch % 2
        q = 1 - p
        if ch + 1 < _NCHUNK:
            h_in[q] = pltpu.async_copy(
                x_hbm.at[b, pl.ds((ch + 1) * _CHS, _CHS)], inbs[q], sins[q]
            )
        h_in[p].wait()
        if ch >= 2:
            h_out[p].wait()
        inb, outb = inbs[p], outbs[p]

        @plsc.parallel_loop(0, _CHS, unroll=4)
        def body(s):
            srow = jnp.full((_L,), s, jnp.int32)
            for g in range(_NG):
                vals = plsc.load_gather(inb, [srow, idx16[g]])
                outb[s, pl.ds(_L * g, _L)] = vals
        h_out[p] = pltpu.async_copy(
            outb, out_hbm.at[b, pl.ds(ch * _CHS, _CHS)], souts[p]
        )
    h_out[(_NCHUNK - 2) % 2].wait()
    h_out[(_NCHUNK - 1) % 2].wait()


def kernel(x):
    x3 = jnp.transpose(x.reshape(B, C, S), (0, 2, 1))
    lidx = _stats(x3).reshape(B, C)
    out3 = _permute(x3, lidx)
    return jnp.transpose(out3, (0, 2, 1)).reshape(B, C, H, W)


# speedup vs baseline: 1.3433x; 1.1343x over previous
import functools

import numpy as np
import jax
import jax.numpy as jnp
from jax import lax
from jax.experimental import pallas as pl
from jax.experimental.pallas import tpu as pltpu
from jax.experimental.pallas import tpu_sc as plsc

B, C, H, W = 32, 384, 24, 24
S = H * W
N = B * C
_RECIP = np.float32(1.0 / 576.0)


def _stats_kernel(y3_ref, gidx_ref):
    for blk in range(2):
        xb = y3_ref[blk]

        acc = None
        for k in range(3):
            for t in range(24):
                row = 24 * t + 8 * k
                sl = xb[row : row + 8, :]
                acc = sl if acc is None else acc + sl
        u0 = acc[0:1, :] + acc[4:5, :]
        u2 = acc[2:3, :] + acc[6:7, :]
        u1 = acc[1:2, :] + acc[5:6, :]
        u3 = acc[3:4, :] + acc[7:8, :]
        total = (u0 + u2) + (u1 + u3)
        m = total * _RECIP

        ik = lax.bitcast_convert_type(m, jnp.int32)
        key = jnp.where(ik >= 0, ik, ik ^ jnp.int32(0x7FFFFFFF))
        key_t = jnp.transpose(key)

        ii = lax.broadcasted_iota(jnp.int32, (C, C), 0)
        jj = lax.broadcasted_iota(jnp.int32, (C, C), 1)
        kj = jnp.broadcast_to(key_t, (C, C))
        ki = jnp.broadcast_to(key, (C, C))
        contrib = (kj > ki) | ((kj == ki) & (ii < jj))
        rank = jnp.sum(contrib.astype(jnp.int32), axis=0, keepdims=True)

        rank_t = jnp.transpose(rank)
        onehot = jnp.broadcast_to(rank_t, (C, C)) == jj
        idx = jnp.sum(jnp.where(onehot, ii, 0), axis=0, keepdims=True)
        gidx_ref[blk] = idx


_stats = pl.pallas_call(
    _stats_kernel,
    grid=(B // 2,),
    in_specs=[pl.BlockSpec((2, S, C), lambda b: (b, 0, 0))],
    out_specs=pl.BlockSpec((2, 1, C), lambda b: (b, 0, 0)),
    out_shape=jax.ShapeDtypeStruct((B, 1, C), jnp.int32),
)

_info = plsc.get_sparse_core_info()
_NC = _info.num_cores
_NW = _info.num_cores * _info.num_subcores
_L = _info.num_lanes
_CHS = 64
_NCHUNK = S // _CHS
_NG = C // _L


@functools.partial(
    pl.kernel,
    mesh=plsc.VectorSubcoreMesh(core_axis_name="c", subcore_axis_name="s"),
    compiler_params=pltpu.CompilerParams(needs_layout_passes=False),
    out_type=jax.ShapeDtypeStruct((B, S, C), jnp.float32),
    scratch_types=[
        pltpu.VMEM((1, C), jnp.int32),
        pltpu.VMEM((_CHS, C), jnp.float32),
        pltpu.VMEM((_CHS, C), jnp.float32),
        pltpu.VMEM((_CHS, C), jnp.float32),
        pltpu.VMEM((_CHS, C), jnp.float32),
        pltpu.SemaphoreType.DMA,
        pltpu.SemaphoreType.DMA,
        pltpu.SemaphoreType.DMA,
        pltpu.SemaphoreType.DMA,
    ],
)
def _permute(x_hbm, idx_hbm, out_hbm, idx_v, inb0, inb1, outb0, outb1,
             sin0, sin1, sout0, sout1):
    b = lax.axis_index("s") * _NC + lax.axis_index("c")
    pltpu.sync_copy(idx_hbm.at[pl.ds(b, 1)], idx_v)
    idx16 = [idx_v[0, pl.ds(_L * g, _L)] for g in range(_NG)]
    inbs, outbs = [inb0, inb1], [outb0, outb1]
    sins, souts = [sin0, sin1], [sout0, sout1]
    h_in = [None, None]
    h_out = [None, None]
    h_in[0] = pltpu.async_copy(x_hbm.at[b, pl.ds(0, _CHS)], inbs[0], sins[0])
    for ch in range(_NCHUNK):
        p = ch % 2
        q = 1 - p
        if ch + 1 < _NCHUNK:
            h_in[q] = pltpu.async_copy(
                x_hbm.at[b, pl.ds((ch + 1) * _CHS, _CHS)], inbs[q], sins[q]
            )
        h_in[p].wait()
        if ch >= 2:
            h_out[p].wait()
        inb, outb = inbs[p], outbs[p]

        @plsc.parallel_loop(0, _CHS, unroll=4)
        def body(s):
            srow = jnp.full((_L,), s, jnp.int32)
            for g in range(_NG):
                vals = plsc.load_gather(inb, [srow, idx16[g]])
                outb[s, pl.ds(_L * g, _L)] = vals
        h_out[p] = pltpu.async_copy(
            outb, out_hbm.at[b, pl.ds(ch * _CHS, _CHS)], souts[p]
        )
    h_out[(_NCHUNK - 2) % 2].wait()
    h_out[(_NCHUNK - 1) % 2].wait()


def kernel(x):
    x3 = jnp.transpose(x.reshape(B, C, S), (0, 2, 1))
    lidx = _stats(x3).reshape(B, C)
    out3 = _permute(x3, lidx)
    return jnp.transpose(out3, (0, 2, 1)).reshape(B, C, H, W)


# speedup vs baseline: 1.4277x; 1.0628x over previous
import functools

import numpy as np
import jax
import jax.numpy as jnp
from jax import lax
from jax.experimental import pallas as pl
from jax.experimental.pallas import tpu as pltpu
from jax.experimental.pallas import tpu_sc as plsc

B, C, H, W = 32, 384, 24, 24
S = H * W
N = B * C
_RECIP = np.float32(1.0 / 576.0)


def _stats_kernel(y3_ref, gidx_ref):
    for blk in range(4):
        xb = y3_ref[blk]

        acc = None
        for k in range(3):
            for t in range(24):
                row = 24 * t + 8 * k
                sl = xb[row : row + 8, :]
                acc = sl if acc is None else acc + sl
        u0 = acc[0:1, :] + acc[4:5, :]
        u2 = acc[2:3, :] + acc[6:7, :]
        u1 = acc[1:2, :] + acc[5:6, :]
        u3 = acc[3:4, :] + acc[7:8, :]
        total = (u0 + u2) + (u1 + u3)
        m = total * _RECIP

        ik = lax.bitcast_convert_type(m, jnp.int32)
        key = jnp.where(ik >= 0, ik, ik ^ jnp.int32(0x7FFFFFFF))
        key_t = jnp.transpose(key)

        ii = lax.broadcasted_iota(jnp.int32, (C, C), 0)
        jj = lax.broadcasted_iota(jnp.int32, (C, C), 1)
        kj = jnp.broadcast_to(key_t, (C, C))
        ki = jnp.broadcast_to(key, (C, C))
        contrib = (kj > ki) | ((kj == ki) & (ii < jj))
        rank = jnp.sum(contrib.astype(jnp.int32), axis=0, keepdims=True)

        rank_t = jnp.transpose(rank)
        onehot = jnp.broadcast_to(rank_t, (C, C)) == jj
        idx = jnp.sum(jnp.where(onehot, ii, 0), axis=0, keepdims=True)
        gidx_ref[blk] = idx


_stats = pl.pallas_call(
    _stats_kernel,
    grid=(B // 4,),
    in_specs=[pl.BlockSpec((4, S, C), lambda b: (b, 0, 0))],
    out_specs=pl.BlockSpec((4, 1, C), lambda b: (b, 0, 0)),
    out_shape=jax.ShapeDtypeStruct((B, 1, C), jnp.int32),
)

_info = plsc.get_sparse_core_info()
_NC = _info.num_cores
_NW = _info.num_cores * _info.num_subcores
_L = _info.num_lanes
_CHS = 64
_NCHUNK = S // _CHS
_NG = C // _L


@functools.partial(
    pl.kernel,
    mesh=plsc.VectorSubcoreMesh(core_axis_name="c", subcore_axis_name="s"),
    compiler_params=pltpu.CompilerParams(needs_layout_passes=False),
    out_type=jax.ShapeDtypeStruct((B, S, C), jnp.float32),
    scratch_types=[
        pltpu.VMEM((1, C), jnp.int32),
        pltpu.VMEM((_CHS, C), jnp.float32),
        pltpu.VMEM((_CHS, C), jnp.float32),
        pltpu.VMEM((_CHS, C), jnp.float32),
        pltpu.VMEM((_CHS, C), jnp.float32),
        pltpu.SemaphoreType.DMA,
        pltpu.SemaphoreType.DMA,
        pltpu.SemaphoreType.DMA,
        pltpu.SemaphoreType.DMA,
    ],
)
def _permute(x_hbm, idx_hbm, out_hbm, idx_v, inb0, inb1, outb0, outb1,
             sin0, sin1, sout0, sout1):
    b = lax.axis_index("s") * _NC + lax.axis_index("c")
    pltpu.sync_copy(idx_hbm.at[pl.ds(b, 1)], idx_v)
    idx16 = [idx_v[0, pl.ds(_L * g, _L)] for g in range(_NG)]
    inbs, outbs = [inb0, inb1], [outb0, outb1]
    sins, souts = [sin0, sin1], [sout0, sout1]
    h_in = [None, None]
    h_out = [None, None]
    h_in[0] = pltpu.async_copy(x_hbm.at[b, pl.ds(0, _CHS)], inbs[0], sins[0])
    for ch in range(_NCHUNK):
        p = ch % 2
        q = 1 - p
        if ch + 1 < _NCHUNK:
            h_in[q] = pltpu.async_copy(
                x_hbm.at[b, pl.ds((ch + 1) * _CHS, _CHS)], inbs[q], sins[q]
            )
        h_in[p].wait()
        if ch >= 2:
            h_out[p].wait()
        inb, outb = inbs[p], outbs[p]

        @plsc.parallel_loop(0, _CHS, unroll=4)
        def body(s):
            srow = jnp.full((_L,), s, jnp.int32)
            for g in range(_NG):
                vals = plsc.load_gather(inb, [srow, idx16[g]])
                outb[s, pl.ds(_L * g, _L)] = vals
        h_out[p] = pltpu.async_copy(
            outb, out_hbm.at[b, pl.ds(ch * _CHS, _CHS)], souts[p]
        )
    h_out[(_NCHUNK - 2) % 2].wait()
    h_out[(_NCHUNK - 1) % 2].wait()


def kernel(x):
    x3 = jnp.transpose(x.reshape(B, C, S), (0, 2, 1))
    lidx = _stats(x3).reshape(B, C)
    out3 = _permute(x3, lidx)
    return jnp.transpose(out3, (0, 2, 1)).reshape(B, C, H, W)


# speedup vs baseline: 1.4487x; 1.0147x over previous
import functools

import numpy as np
import jax
import jax.numpy as jnp
from jax import lax
from jax.experimental import pallas as pl
from jax.experimental.pallas import tpu as pltpu
from jax.experimental.pallas import tpu_sc as plsc

B, C, H, W = 32, 384, 24, 24
S = H * W
N = B * C
_RECIP = np.float32(1.0 / 576.0)


def _stats_kernel(y3_ref, gidx_ref):
    for blk in range(8):
        xb = y3_ref[blk]

        acc = None
        for k in range(3):
            for t in range(24):
                row = 24 * t + 8 * k
                sl = xb[row : row + 8, :]
                acc = sl if acc is None else acc + sl
        u0 = acc[0:1, :] + acc[4:5, :]
        u2 = acc[2:3, :] + acc[6:7, :]
        u1 = acc[1:2, :] + acc[5:6, :]
        u3 = acc[3:4, :] + acc[7:8, :]
        total = (u0 + u2) + (u1 + u3)
        m = total * _RECIP

        ik = lax.bitcast_convert_type(m, jnp.int32)
        key = jnp.where(ik >= 0, ik, ik ^ jnp.int32(0x7FFFFFFF))
        key_t = jnp.transpose(key)

        ii = lax.broadcasted_iota(jnp.int32, (C, C), 0)
        jj = lax.broadcasted_iota(jnp.int32, (C, C), 1)
        kj = jnp.broadcast_to(key_t, (C, C))
        ki = jnp.broadcast_to(key, (C, C))
        contrib = (kj > ki) | ((kj == ki) & (ii < jj))
        rank = jnp.sum(contrib.astype(jnp.int32), axis=0, keepdims=True)

        rank_t = jnp.transpose(rank)
        onehot = jnp.broadcast_to(rank_t, (C, C)) == jj
        idx = jnp.sum(jnp.where(onehot, ii, 0), axis=0, keepdims=True)
        gidx_ref[blk] = idx


_stats = pl.pallas_call(
    _stats_kernel,
    grid=(B // 8,),
    in_specs=[pl.BlockSpec((8, S, C), lambda b: (b, 0, 0))],
    out_specs=pl.BlockSpec((8, 1, C), lambda b: (b, 0, 0)),
    out_shape=jax.ShapeDtypeStruct((B, 1, C), jnp.int32),
)

_info = plsc.get_sparse_core_info()
_NC = _info.num_cores
_NW = _info.num_cores * _info.num_subcores
_L = _info.num_lanes
_CHS = 64
_NCHUNK = S // _CHS
_NG = C // _L


@functools.partial(
    pl.kernel,
    mesh=plsc.VectorSubcoreMesh(core_axis_name="c", subcore_axis_name="s"),
    compiler_params=pltpu.CompilerParams(needs_layout_passes=False),
    out_type=jax.ShapeDtypeStruct((B, S, C), jnp.float32),
    scratch_types=[
        pltpu.VMEM((1, C), jnp.int32),
        pltpu.VMEM((_CHS, C), jnp.float32),
        pltpu.VMEM((_CHS, C), jnp.float32),
        pltpu.VMEM((_CHS, C), jnp.float32),
        pltpu.VMEM((_CHS, C), jnp.float32),
        pltpu.SemaphoreType.DMA,
        pltpu.SemaphoreType.DMA,
        pltpu.SemaphoreType.DMA,
        pltpu.SemaphoreType.DMA,
    ],
)
def _permute(x_hbm, idx_hbm, out_hbm, idx_v, inb0, inb1, outb0, outb1,
             sin0, sin1, sout0, sout1):
    b = lax.axis_index("s") * _NC + lax.axis_index("c")
    pltpu.sync_copy(idx_hbm.at[pl.ds(b, 1)], idx_v)
    idx16 = [idx_v[0, pl.ds(_L * g, _L)] for g in range(_NG)]
    inbs, outbs = [inb0, inb1], [outb0, outb1]
    sins, souts = [sin0, sin1], [sout0, sout1]
    h_in = [None, None]
    h_out = [None, None]
    h_in[0] = pltpu.async_copy(x_hbm.at[b, pl.ds(0, _CHS)], inbs[0], sins[0])
    for ch in range(_NCHUNK):
        p = ch % 2
        q = 1 - p
        if ch + 1 < _NCHUNK:
            h_in[q] = pltpu.async_copy(
                x_hbm.at[b, pl.ds((ch + 1) * _CHS, _CHS)], inbs[q], sins[q]
            )
        h_in[p].wait()
        if ch >= 2:
            h_out[p].wait()
        inb, outb = inbs[p], outbs[p]

        @plsc.parallel_loop(0, _CHS, unroll=4)
        def body(s):
            srow = jnp.full((_L,), s, jnp.int32)
            for g in range(_NG):
                vals = plsc.load_gather(inb, [srow, idx16[g]])
                outb[s, pl.ds(_L * g, _L)] = vals
        h_out[p] = pltpu.async_copy(
            outb, out_hbm.at[b, pl.ds(ch * _CHS, _CHS)], souts[p]
        )
    h_out[(_NCHUNK - 2) % 2].wait()
    h_out[(_NCHUNK - 1) % 2].wait()


def kernel(x):
    x3 = jnp.transpose(x.reshape(B, C, S), (0, 2, 1))
    lidx = _stats(x3).reshape(B, C)
    out3 = _permute(x3, lidx)
    return jnp.transpose(out3, (0, 2, 1)).reshape(B, C, H, W)


# speedup vs baseline: 1.4630x; 1.0099x over previous
import functools

import numpy as np
import jax
import jax.numpy as jnp
from jax import lax
from jax.experimental import pallas as pl
from jax.experimental.pallas import tpu as pltpu
from jax.experimental.pallas import tpu_sc as plsc

B, C, H, W = 32, 384, 24, 24
S = H * W
N = B * C
_RECIP = np.float32(1.0 / 576.0)


def _stats_kernel(y3_ref, gidx_ref):
    for blk in range(8):
        xb = y3_ref[blk]

        acc = None
        for k in range(3):
            for t in range(24):
                row = 24 * t + 8 * k
                sl = xb[row : row + 8, :]
                acc = sl if acc is None else acc + sl
        u0 = acc[0:1, :] + acc[4:5, :]
        u2 = acc[2:3, :] + acc[6:7, :]
        u1 = acc[1:2, :] + acc[5:6, :]
        u3 = acc[3:4, :] + acc[7:8, :]
        total = (u0 + u2) + (u1 + u3)
        m = total * _RECIP

        ik = lax.bitcast_convert_type(m, jnp.int32)
        key = jnp.where(ik >= 0, ik, ik ^ jnp.int32(0x7FFFFFFF))
        key_t = jnp.transpose(key)

        ii = lax.broadcasted_iota(jnp.int32, (C, C), 0)
        jj = lax.broadcasted_iota(jnp.int32, (C, C), 1)
        kj = jnp.broadcast_to(key_t, (C, C))
        ki = jnp.broadcast_to(key, (C, C))
        contrib = (kj > ki) | ((kj == ki) & (ii < jj))
        rank = jnp.sum(contrib.astype(jnp.int32), axis=0, keepdims=True)

        rank_t = jnp.transpose(rank)
        onehot = jnp.broadcast_to(rank_t, (C, C)) == jj
        idx = jnp.sum(jnp.where(onehot, ii, 0), axis=0, keepdims=True)
        gidx_ref[blk] = idx


_stats = pl.pallas_call(
    _stats_kernel,
    grid=(B // 8,),
    in_specs=[pl.BlockSpec((8, S, C), lambda b: (b, 0, 0))],
    out_specs=pl.BlockSpec((8, 1, C), lambda b: (b, 0, 0)),
    out_shape=jax.ShapeDtypeStruct((B, 1, C), jnp.int32),
)

_info = plsc.get_sparse_core_info()
_NC = _info.num_cores
_NW = _info.num_cores * _info.num_subcores
_L = _info.num_lanes
_CHS = 72
_NCHUNK = S // _CHS
_NG = C // _L


@functools.partial(
    pl.kernel,
    mesh=plsc.VectorSubcoreMesh(core_axis_name="c", subcore_axis_name="s"),
    compiler_params=pltpu.CompilerParams(needs_layout_passes=False),
    out_type=jax.ShapeDtypeStruct((B, S, C), jnp.float32),
    scratch_types=[
        pltpu.VMEM((1, C), jnp.int32),
        pltpu.VMEM((_CHS, C), jnp.float32),
        pltpu.VMEM((_CHS, C), jnp.float32),
        pltpu.VMEM((_CHS, C), jnp.float32),
        pltpu.VMEM((_CHS, C), jnp.float32),
        pltpu.SemaphoreType.DMA,
        pltpu.SemaphoreType.DMA,
        pltpu.SemaphoreType.DMA,
        pltpu.SemaphoreType.DMA,
    ],
)
def _permute(x_hbm, idx_hbm, out_hbm, idx_v, inb0, inb1, outb0, outb1,
             sin0, sin1, sout0, sout1):
    b = lax.axis_index("s") * _NC + lax.axis_index("c")
    pltpu.sync_copy(idx_hbm.at[pl.ds(b, 1)], idx_v)
    idx16 = [idx_v[0, pl.ds(_L * g, _L)] for g in range(_NG)]
    inbs, outbs = [inb0, inb1], [outb0, outb1]
    sins, souts = [sin0, sin1], [sout0, sout1]
    h_in = [None, None]
    h_out = [None, None]
    h_in[0] = pltpu.async_copy(x_hbm.at[b, pl.ds(0, _CHS)], inbs[0], sins[0])
    for ch in range(_NCHUNK):
        p = ch % 2
        q = 1 - p
        if ch + 1 < _NCHUNK:
            h_in[q] = pltpu.async_copy(
                x_hbm.at[b, pl.ds((ch + 1) * _CHS, _CHS)], inbs[q], sins[q]
            )
        h_in[p].wait()
        if ch >= 2:
            h_out[p].wait()
        inb, outb = inbs[p], outbs[p]

        @plsc.parallel_loop(0, _CHS, unroll=4)
        def body(s):
            srow = jnp.full((_L,), s, jnp.int32)
            for g in range(_NG):
                vals = plsc.load_gather(inb, [srow, idx16[g]])
                outb[s, pl.ds(_L * g, _L)] = vals
        h_out[p] = pltpu.async_copy(
            outb, out_hbm.at[b, pl.ds(ch * _CHS, _CHS)], souts[p]
        )
    h_out[(_NCHUNK - 2) % 2].wait()
    h_out[(_NCHUNK - 1) % 2].wait()


def kernel(x):
    x3 = jnp.transpose(x.reshape(B, C, S), (0, 2, 1))
    lidx = _stats(x3).reshape(B, C)
    out3 = _permute(x3, lidx)
    return jnp.transpose(out3, (0, 2, 1)).reshape(B, C, H, W)


# speedup vs baseline: 1.4860x; 1.0157x over previous
import functools

import numpy as np
import jax
import jax.numpy as jnp
from jax import lax
from jax.experimental import pallas as pl
from jax.experimental.pallas import tpu as pltpu
from jax.experimental.pallas import tpu_sc as plsc

B, C, H, W = 32, 384, 24, 24
S = H * W
N = B * C
_RECIP = np.float32(1.0 / 576.0)


def _stats_kernel(y3_ref, gidx_ref):
    for blk in range(8):
        xb = y3_ref[blk]

        acc = None
        for k in range(3):
            for t in range(24):
                row = 24 * t + 8 * k
                sl = xb[row : row + 8, :]
                acc = sl if acc is None else acc + sl
        u0 = acc[0:1, :] + acc[4:5, :]
        u2 = acc[2:3, :] + acc[6:7, :]
        u1 = acc[1:2, :] + acc[5:6, :]
        u3 = acc[3:4, :] + acc[7:8, :]
        total = (u0 + u2) + (u1 + u3)
        m = total * _RECIP

        ik = lax.bitcast_convert_type(m, jnp.int32)
        key = jnp.where(ik >= 0, ik, ik ^ jnp.int32(0x7FFFFFFF))
        key_t = jnp.transpose(key)

        ii = lax.broadcasted_iota(jnp.int32, (C, C), 0)
        jj = lax.broadcasted_iota(jnp.int32, (C, C), 1)
        kj = jnp.broadcast_to(key_t, (C, C))
        ki = jnp.broadcast_to(key, (C, C))
        contrib = (kj > ki) | ((kj == ki) & (ii < jj))
        rank = jnp.sum(contrib.astype(jnp.int32), axis=0, keepdims=True)

        rank_t = jnp.transpose(rank)
        onehot = jnp.broadcast_to(rank_t, (C, C)) == jj
        idx = jnp.sum(jnp.where(onehot, ii, 0), axis=0, keepdims=True)
        gidx_ref[blk] = idx


_stats = pl.pallas_call(
    _stats_kernel,
    grid=(B // 8,),
    in_specs=[pl.BlockSpec((8, S, C), lambda b: (b, 0, 0))],
    out_specs=pl.BlockSpec((8, 1, C), lambda b: (b, 0, 0)),
    out_shape=jax.ShapeDtypeStruct((B, 1, C), jnp.int32),
)

_info = plsc.get_sparse_core_info()
_NC = _info.num_cores
_NW = _info.num_cores * _info.num_subcores
_L = _info.num_lanes
_CHS = 72
_NCHUNK = S // _CHS
_NG = C // _L


@functools.partial(
    pl.kernel,
    mesh=plsc.VectorSubcoreMesh(core_axis_name="c", subcore_axis_name="s"),
    compiler_params=pltpu.CompilerParams(needs_layout_passes=False),
    out_type=jax.ShapeDtypeStruct((B, S, C), jnp.float32),
    scratch_types=[
        pltpu.VMEM((1, C), jnp.int32),
        pltpu.VMEM((_CHS, C), jnp.float32),
        pltpu.VMEM((_CHS, C), jnp.float32),
        pltpu.VMEM((_CHS, C), jnp.float32),
        pltpu.VMEM((_CHS, C), jnp.float32),
        pltpu.SemaphoreType.DMA,
        pltpu.SemaphoreType.DMA,
        pltpu.SemaphoreType.DMA,
        pltpu.SemaphoreType.DMA,
    ],
)
def _permute(x_hbm, idx_hbm, out_hbm, idx_v, inb0, inb1, outb0, outb1,
             sin0, sin1, sout0, sout1):
    b = lax.axis_index("s") * _NC + lax.axis_index("c")
    pltpu.sync_copy(idx_hbm.at[pl.ds(b, 1)], idx_v)
    idx16 = [idx_v[0, pl.ds(_L * g, _L)] for g in range(_NG)]
    inbs, outbs = [inb0, inb1], [outb0, outb1]
    sins, souts = [sin0, sin1], [sout0, sout1]
    h_in = [None, None]
    h_out = [None, None]
    h_in[0] = pltpu.async_copy(x_hbm.at[b, pl.ds(0, _CHS)], inbs[0], sins[0])
    for ch in range(_NCHUNK):
        p = ch % 2
        q = 1 - p
        if ch + 1 < _NCHUNK:
            h_in[q] = pltpu.async_copy(
                x_hbm.at[b, pl.ds((ch + 1) * _CHS, _CHS)], inbs[q], sins[q]
            )
        h_in[p].wait()
        if ch >= 2:
            h_out[p].wait()
        inb, outb = inbs[p], outbs[p]

        @plsc.parallel_loop(0, _CHS, unroll=6)
        def body(s):
            srow = jnp.full((_L,), s, jnp.int32)
            for g in range(_NG):
                vals = plsc.load_gather(inb, [srow, idx16[g]])
                outb[s, pl.ds(_L * g, _L)] = vals
        h_out[p] = pltpu.async_copy(
            outb, out_hbm.at[b, pl.ds(ch * _CHS, _CHS)], souts[p]
        )
    h_out[(_NCHUNK - 2) % 2].wait()
    h_out[(_NCHUNK - 1) % 2].wait()


def kernel(x):
    x3 = jnp.transpose(x.reshape(B, C, S), (0, 2, 1))
    lidx = _stats(x3).reshape(B, C)
    out3 = _permute(x3, lidx)
    return jnp.transpose(out3, (0, 2, 1)).reshape(B, C, H, W)
